# baseline (speedup 1.0000x reference)
import functools

import jax
import jax.numpy as jnp
from jax import lax
from jax.experimental import pallas as pl
from jax.experimental.pallas import tpu as pltpu
from jax.experimental.pallas import tpu_sc as plsc

NC = 2
NS = 16
NW = NC * NS
L = 16

VOCAB = 1000000
DIM = 64
PAD = 128
MET_ROW = 5

B_SEQ = 4096
L_SEQ = 200
N = B_SEQ * L_SEQ
B_PER_W = N // NW
CHUNK = 256
GSZ = 128
N_CHUNKS = B_PER_W // CHUNK
QUARTERS = DIM // L

TBLK = 512


def _transpose_body(tt_ref, y_ref):
    t = tt_ref[...].T
    y_ref[...] = jnp.concatenate([t, t], axis=1)


@jax.jit
def _stage_table(table_t):
    grid = (VOCAB + TBLK - 1) // TBLK
    return pl.pallas_call(
        _transpose_body,
        grid=(grid,),
        in_specs=[pl.BlockSpec((DIM, TBLK), lambda i: (0, i))],
        out_specs=pl.BlockSpec((TBLK, PAD), lambda i: (i, 0)),
        out_shape=jax.ShapeDtypeStruct((VOCAB, PAD), jnp.float32),
    )(table_t)


def _body(table_hbm, seq_hbm, met_hbm, out_hbm,
          seq_v, s_v, rows_v, out_v, row5_v, sem):
    wid = lax.axis_index("s") * NC + lax.axis_index("c")
    base0 = wid * B_PER_W

    pltpu.sync_copy(table_hbm.at[pl.ds(MET_ROW, 1), :], row5_v)
    r5 = [row5_v[0, pl.ds(q * L, L)] for q in range(QUARTERS)]

    def chunk_body(ci, _):
        base = base0 + ci * CHUNK
        pltpu.sync_copy(seq_hbm.at[pl.ds(base, CHUNK)], seq_v)
        pltpu.sync_copy(met_hbm.at[pl.ds(base, CHUNK)], s_v)

        def pre(g, _):
            sv = seq_v[pl.ds(g * L, L)]
            mv = s_v[pl.ds(g * L, L)]
            keep = sv != 0
            seq_v[pl.ds(g * L, L)] = jnp.where(keep, sv, MET_ROW)
            s_v[pl.ds(g * L, L)] = jnp.where(keep, mv, -1.0)
            return 0

        lax.fori_loop(0, CHUNK // L, pre, 0, unroll=2)

        copies = [
            pltpu.async_copy(
                table_hbm.at[seq_v.at[pl.ds(t * GSZ, GSZ)]],
                rows_v.at[pl.ds(t * GSZ, GSZ)],
                sem,
            )
            for t in range(CHUNK // GSZ)
        ]
        for cp in copies:
            cp.wait()

        def rowfn(i, _):
            sb = plsc.load_gather(s_v, [jnp.full((L,), i, jnp.int32)])
            for q in range(QUARTERS):
                v = rows_v[i, pl.ds(q * L, L)]
                out_v[i, pl.ds(q * L, L)] = v + sb * r5[q]
            return 0

        lax.fori_loop(0, CHUNK, rowfn, 0, unroll=4)

        pltpu.sync_copy(out_v, out_hbm.at[pl.ds(base, CHUNK), :])
        return 0

    lax.fori_loop(0, N_CHUNKS, chunk_body, 0)


@jax.jit
def _run(table_sc, seq_f, met_f):
    mesh = plsc.VectorSubcoreMesh(
        core_axis_name="c", subcore_axis_name="s",
        num_cores=NC, num_subcores=NS,
    )
    f = pl.kernel(
        _body,
        out_type=jax.ShapeDtypeStruct((N, DIM), jnp.float32),
        mesh=mesh,
        compiler_params=pltpu.CompilerParams(
            needs_layout_passes=False, use_tc_tiling_on_sc=False,
        ),
        scratch_types=[
            pltpu.VMEM((CHUNK,), jnp.int32),
            pltpu.VMEM((CHUNK,), jnp.float32),
            pltpu.VMEM((CHUNK, PAD), jnp.float32),
            pltpu.VMEM((CHUNK, DIM), jnp.float32),
            pltpu.VMEM((1, PAD), jnp.float32),
            pltpu.SemaphoreType.DMA,
        ],
    )
    return f(table_sc, seq_f, met_f)


def kernel(seq, met, table):
    seq_f = seq.reshape(N)
    met_f = met.reshape(N)
    staged = _stage_table(table.T)
    table_sc = staged.reshape(-1).reshape(VOCAB, PAD)
    out = _run(table_sc, seq_f, met_f)
    return out.reshape(B_SEQ, L_SEQ, DIM)


# speedup vs baseline: 1.9838x; 1.9838x over previous
import functools

import jax
import jax.numpy as jnp
from jax import lax
from jax.experimental import pallas as pl
from jax.experimental.pallas import tpu as pltpu
from jax.experimental.pallas import tpu_sc as plsc

NC = 2
NS = 16
NW = NC * NS
L = 16

VOCAB = 1000000
DIM = 64
MET_ROW = 5

B_SEQ = 4096
L_SEQ = 200
N = B_SEQ * L_SEQ
B_PER_W = N // NW
CHUNK = 512
GSZ = 128
N_CHUNKS = B_PER_W // CHUNK
QUARTERS = DIM // L


def _body(table_hbm, seq_hbm, met_hbm, out_hbm,
          seq_v, s_v, rows_v, row5_v, gsems, osems):
    wid = lax.axis_index("s") * NC + lax.axis_index("c")
    base0 = wid * B_PER_W

    pltpu.sync_copy(table_hbm.at[pl.ds(MET_ROW, 1), :], row5_v)
    r5 = [row5_v[0, pl.ds(q * L, L)] for q in range(QUARTERS)]

    pltpu.sync_copy(seq_hbm.at[pl.ds(base0, B_PER_W)], seq_v)
    pltpu.sync_copy(met_hbm.at[pl.ds(base0, B_PER_W)], s_v)

    def pre(g, _):
        sv = seq_v[pl.ds(g * L, L)]
        mv = s_v[pl.ds(g * L, L)]
        keep = sv != 0
        seq_v[pl.ds(g * L, L)] = jnp.where(keep, sv, MET_ROW)
        s_v[pl.ds(g * L, L)] = jnp.where(keep, mv, -1.0)
        return 0

    lax.fori_loop(0, B_PER_W // L, pre, 0, unroll=4)

    def fire_gather(ci, b):
        for t in range(CHUNK // GSZ):
            pltpu.async_copy(
                table_hbm.at[seq_v.at[pl.ds(ci * CHUNK + t * GSZ, GSZ)]],
                rows_v.at[b, pl.ds(t * GSZ, GSZ)],
                gsems.at[b],
            )

    def wait_gather(ci, b):
        for t in range(CHUNK // GSZ):
            pltpu.make_async_copy(
                table_hbm.at[seq_v.at[pl.ds(ci * CHUNK + t * GSZ, GSZ)]],
                rows_v.at[b, pl.ds(t * GSZ, GSZ)],
                gsems.at[b],
            ).wait()

    def out_copy(ci, b):
        return pltpu.make_async_copy(
            rows_v.at[b],
            out_hbm.at[pl.ds(base0 + ci * CHUNK, CHUNK), :],
            osems.at[b],
        )

    fire_gather(0, 0)

    def chunk_body(ci, _):
        b = lax.rem(ci, 2)
        nb = 1 - b

        @pl.when(ci + 1 < N_CHUNKS)
        def _prefetch():
            @pl.when(ci >= 1)
            def _drain_prev_out():
                out_copy(ci - 1, nb).wait()
            fire_gather(ci + 1, nb)

        wait_gather(ci, b)

        def rowfn(i, _):
            sb = plsc.load_gather(
                s_v, [jnp.full((L,), ci * CHUNK + i, jnp.int32)])
            for q in range(QUARTERS):
                v = rows_v[b, i, pl.ds(q * L, L)]
                rows_v[b, i, pl.ds(q * L, L)] = v + sb * r5[q]
            return 0

        lax.fori_loop(0, CHUNK, rowfn, 0, unroll=4)

        out_copy(ci, b).start()
        return 0

    lax.fori_loop(0, N_CHUNKS, chunk_body, 0)

    out_copy(N_CHUNKS - 2, lax.rem(N_CHUNKS - 2, 2)).wait()
    out_copy(N_CHUNKS - 1, lax.rem(N_CHUNKS - 1, 2)).wait()


@jax.jit
def _run(table, seq_f, met_f):
    mesh = plsc.VectorSubcoreMesh(
        core_axis_name="c", subcore_axis_name="s",
        num_cores=NC, num_subcores=NS,
    )
    f = pl.kernel(
        _body,
        out_type=jax.ShapeDtypeStruct((N, DIM), jnp.float32),
        mesh=mesh,
        compiler_params=pltpu.CompilerParams(
            needs_layout_passes=False, use_tc_tiling_on_sc=False,
        ),
        scratch_types=[
            pltpu.VMEM((B_PER_W,), jnp.int32),
            pltpu.VMEM((B_PER_W,), jnp.float32),
            pltpu.VMEM((2, CHUNK, DIM), jnp.float32),
            pltpu.VMEM((1, DIM), jnp.float32),
            pltpu.SemaphoreType.DMA((2,)),
            pltpu.SemaphoreType.DMA((2,)),
        ],
    )
    return f(table, seq_f, met_f)


def kernel(seq, met, table):
    seq_f = seq.reshape(N)
    met_f = met.reshape(N)
    out = _run(table, seq_f, met_f)
    return out.reshape(B_SEQ, L_SEQ, DIM)
